# baseline (speedup 1.0000x reference)
import jax
import jax.numpy as jnp
from jax import lax
from jax.experimental import pallas as pl
from jax.experimental.pallas import tpu as pltpu
from jax.experimental.pallas import tpu_sc as plsc

_T = 8192
_F = 4096
_M = 2 * _T - 1
_NC = 2
_NS = 16
_NTIME = 8
_NFEAT = 4
_FW = _F // _NFEAT
_TW = _T // _NTIME
_K = 16
_KH = _K // 2
_NCHUNK = _TW // _K
_L = 16


def _sc_body(y_hbm, out_hbm, vb0, vb1, vb2, ob0, ob1,
             ls0, ls1, ls2, ss0, ss1):
    wid = lax.axis_index("s") * _NC + lax.axis_index("c")
    tr = wid // _NFEAT
    fc = wid % _NFEAT
    base = tr * _TW
    c0 = fc * _FW
    vbufs = (vb0, vb1, vb2)
    obufs = (ob0, ob1)
    lsems = (ls0, ls1, ls2)
    ssems = (ss0, ss1)

    def issue_load(ci, slot):
        start = pl.multiple_of(jnp.clip(base + ci * _K, 0, _T - _K), _K)
        pltpu.async_copy(y_hbm.at[pl.ds(start, _K), pl.ds(c0, _FW)],
                         vbufs[slot], lsems[slot])

    def wait_load(slot):
        pltpu.make_async_copy(y_hbm.at[pl.ds(0, _K), pl.ds(0, _FW)],
                              vbufs[slot], lsems[slot]).wait()

    def issue_store(ci, b):
        row = pl.multiple_of(2 * (base + ci * _K), 2 * _K)
        pltpu.async_copy(obufs[b],
                         out_hbm.at[pl.ds(row, 2 * _K),
                                    pl.ds(c0, _FW)], ssems[b])

    def wait_store(b):
        pltpu.make_async_copy(obufs[b],
                              out_hbm.at[pl.ds(0, 2 * _K), pl.ds(0, _FW)],
                              ssems[b]).wait()

    def run_pass(pb, ab, nb, ob, m0f, i_lo):
        avs = []
        bvs = []
        for q in range(_KH):
            i = i_lo + q
            a = (m0f + i) * (1.0 / _M)
            bw = (m0f + (i + _T)) * (1.0 / _M)
            avs.append(jnp.broadcast_to(a, (_L,)))
            bvs.append(jnp.broadcast_to(bw, (_L,)))

        @plsc.parallel_loop(0, _FW, _L, unroll=1)
        def _(j):
            lv = []
            for r in range(i_lo - 1, i_lo + _KH + 1):
                if r < 0:
                    lv.append(pb[_K - 1, pl.ds(j, _L)])
                elif r >= _K:
                    lv.append(nb[0, pl.ds(j, _L)])
                else:
                    lv.append(ab[r, pl.ds(j, _L)])
            diff = [lv[r] - lv[r + 1] for r in range(_KH + 1)]
            for q in range(_KH):
                i = i_lo + q
                ob[2 * i, pl.ds(j, _L)] = lv[q + 1] + avs[q] * diff[q]
                ob[2 * i + 1, pl.ds(j, _L)] = lv[q + 2] + bvs[q] * diff[q + 1]

    issue_load(-1, 2)
    issue_load(0, 0)
    issue_load(1, 1)
    wait_load(2)
    wait_load(0)

    def chunk_step(ci, slots, b):
        sp, sa, sn = slots
        wait_load(sn)

        @pl.when(ci >= 2)
        def _():
            wait_store(b)

        m0f = (base + ci * _K).astype(jnp.float32)
        pbuf, abuf, nbuf = vbufs[sp], vbufs[sa], vbufs[sn]
        obuf = obufs[b]
        run_pass(pbuf, abuf, nbuf, obuf, m0f, 0)
        run_pass(pbuf, abuf, nbuf, obuf, m0f, _KH)
        issue_store(ci, b)

        @pl.when(ci + 2 <= _NCHUNK)
        def _():
            issue_load(ci + 2, sp)

    def six_body(g, carry):
        for ph in range(6):
            ci = 6 * g + ph
            chunk_step(ci, ((2 + ph) % 3, (0 + ph) % 3, (1 + ph) % 3),
                       ph % 2)
        return carry

    lax.fori_loop(0, _NCHUNK // 6, six_body, 0)
    for ph in range(_NCHUNK % 6):
        ci = (_NCHUNK // 6) * 6 + ph
        chunk_step(jnp.int32(ci), ((2 + ph) % 3, (0 + ph) % 3, (1 + ph) % 3),
                   ph % 2)

    wait_store(0)
    wait_store(1)


def kernel(y):
    y2 = y.reshape(_T, _F)
    mesh = plsc.VectorSubcoreMesh(core_axis_name="c", subcore_axis_name="s")
    out = pl.kernel(
        _sc_body,
        mesh=mesh,
        out_type=jax.ShapeDtypeStruct((2 * _T, _F), jnp.float32),
        scratch_types=[
            pltpu.VMEM((_K, _FW), jnp.float32),
            pltpu.VMEM((_K, _FW), jnp.float32),
            pltpu.VMEM((_K, _FW), jnp.float32),
            pltpu.VMEM((2 * _K, _FW), jnp.float32),
            pltpu.VMEM((2 * _K, _FW), jnp.float32),
            pltpu.SemaphoreType.DMA,
            pltpu.SemaphoreType.DMA,
            pltpu.SemaphoreType.DMA,
            pltpu.SemaphoreType.DMA,
            pltpu.SemaphoreType.DMA,
        ],
    )(y2)
    return out.reshape(2 * _T, 4, 1024)


# speedup vs baseline: 3.3113x; 3.3113x over previous
import jax
import jax.numpy as jnp
from jax import lax
from jax.experimental import pallas as pl
from jax.experimental.pallas import tpu as pltpu
from jax.experimental.pallas import tpu_sc as plsc

_T = 8192
_B = 4
_C = 1024
_M = 2 * _T - 1
_NC = 2
_NS = 16
_NW = _NC * _NS
_TW = _T // _NW
_K = 4
_NCHUNK = _TW // _K
_L = 16
_NPAIR = _NCHUNK // 2


def _sc_body(y_hbm, out_hbm, vb0, vb1, ob0, ob1, ls0, ls1, ss0, ss1):
    wid = lax.axis_index("s") * _NC + lax.axis_index("c")
    base = wid * _TW
    vbufs = (vb0, vb1)
    obufs = (ob0, ob1)
    lsems = (ls0, ls1)
    ssems = (ss0, ss1)

    def issue_load(ci, b):
        m0 = base + ci * _K
        prow = jnp.maximum(m0 - 1, 0)
        nrow = jnp.minimum(m0 + _K, _T - 1)
        pltpu.async_copy(y_hbm.at[pl.ds(prow, 1)],
                         vbufs[b].at[pl.ds(0, 1)], lsems[b])
        pltpu.async_copy(y_hbm.at[pl.ds(m0, _K)],
                         vbufs[b].at[pl.ds(1, _K)], lsems[b])
        pltpu.async_copy(y_hbm.at[pl.ds(nrow, 1)],
                         vbufs[b].at[pl.ds(_K + 1, 1)], lsems[b])

    def wait_load(b):
        pltpu.make_async_copy(y_hbm.at[pl.ds(0, _K + 2)],
                              vbufs[b], lsems[b]).wait()

    def issue_store(ci, b):
        m0 = base + ci * _K
        pltpu.async_copy(obufs[b], out_hbm.at[pl.ds(2 * m0, 2 * _K)],
                         ssems[b])

    def wait_store(b):
        pltpu.make_async_copy(obufs[b], out_hbm.at[pl.ds(0, 2 * _K)],
                              ssems[b]).wait()

    def compute(ci, b):
        m0f = (base + ci * _K).astype(jnp.float32)
        avs = []
        bvs = []
        for i in range(_K):
            a = (m0f + i) * (1.0 / _M)
            bw = (m0f + (i + _T)) * (1.0 / _M)
            avs.append(jnp.broadcast_to(a, (_L,)))
            bvs.append(jnp.broadcast_to(bw, (_L,)))
        vb = vbufs[b]
        ob = obufs[b]

        @plsc.parallel_loop(0, _C, _L, unroll=1)
        def _(j):
            for s in range(_B):
                lv = [vb[r, s, pl.ds(j, _L)] for r in range(_K + 2)]
                diff = [lv[r] - lv[r + 1] for r in range(_K + 1)]
                for i in range(_K):
                    ob[2 * i, s, pl.ds(j, _L)] = lv[i + 1] + avs[i] * diff[i]
                    ob[2 * i + 1, s, pl.ds(j, _L)] = (
                        lv[i + 2] + bvs[i] * diff[i + 1])

    issue_load(0, 0)
    issue_load(1, 1)

    def pair_body(g, carry):
        for b in range(2):
            ci = 2 * g + b
            wait_load(b)

            @pl.when(g >= 1)
            def _():
                wait_store(b)

            compute(ci, b)
            issue_store(ci, b)

            @pl.when(g <= _NPAIR - 2)
            def _():
                issue_load(ci + 2, b)

        return carry

    lax.fori_loop(0, _NPAIR, pair_body, 0)
    wait_store(0)
    wait_store(1)


def kernel(y):
    mesh = plsc.VectorSubcoreMesh(core_axis_name="c", subcore_axis_name="s")
    return pl.kernel(
        _sc_body,
        mesh=mesh,
        out_type=jax.ShapeDtypeStruct((2 * _T, _B, _C), jnp.float32),
        scratch_types=[
            pltpu.VMEM((_K + 2, _B, _C), jnp.float32),
            pltpu.VMEM((_K + 2, _B, _C), jnp.float32),
            pltpu.VMEM((2 * _K, _B, _C), jnp.float32),
            pltpu.VMEM((2 * _K, _B, _C), jnp.float32),
            pltpu.SemaphoreType.DMA,
            pltpu.SemaphoreType.DMA,
            pltpu.SemaphoreType.DMA,
            pltpu.SemaphoreType.DMA,
        ],
    )(y)
